# baseline (speedup 1.0000x reference)
import jax
import jax.numpy as jnp
from jax import lax
from jax.experimental import pallas as pl
from jax.experimental.pallas import tpu as pltpu
from jax.experimental.pallas import tpu_sc as plsc

_B = 4096
_D = 128
_NC = 2
_NS = 16
_NW = _NC * _NS
_BPW = _B // _NW
_L = 16

_DNUMS = lax.GatherDimensionNumbers(
    offset_dims=(), collapsed_slice_dims=(0,), start_index_map=(0,))


def _dyn_gather(x, idx):
    return lax.gather(x, idx[:, None], _DNUMS, slice_sizes=(1,),
                      mode=lax.GatherScatterMode.PROMISE_IN_BOUNDS)


def _body(idx_hbm, table_hbm, left_hbm, right_hbm,
          ent_v, lidx_v, ridx_v, lrows_v, rrows_v, sem):
    wid = lax.axis_index("s") * _NC + lax.axis_index("c")
    pltpu.sync_copy(idx_hbm.at[wid], ent_v)

    lanes = lax.iota(jnp.int32, _L)
    e_idx = (2 * lanes) % _L
    o_idx = (2 * lanes + 1) % _L
    low = lanes < (_L // 2)
    for j in range(_BPW // _L):
        a = ent_v[pl.ds(2 * _L * j, _L)]
        b = ent_v[pl.ds(2 * _L * j + _L, _L)]
        lidx_v[pl.ds(_L * j, _L)] = jnp.where(
            low, _dyn_gather(a, e_idx), _dyn_gather(b, e_idx))
        ridx_v[pl.ds(_L * j, _L)] = jnp.where(
            low, _dyn_gather(a, o_idx), _dyn_gather(b, o_idx))

    cl = pltpu.async_copy(table_hbm.at[lidx_v], lrows_v, sem)
    cr = pltpu.async_copy(table_hbm.at[ridx_v], rrows_v, sem)
    cl.wait()
    cr.wait()

    base = wid * _BPW
    pltpu.sync_copy(lrows_v, left_hbm.at[pl.ds(base, _BPW)])
    pltpu.sync_copy(rrows_v, right_hbm.at[pl.ds(base, _BPW)])


_gather = pl.kernel(
    _body,
    out_type=(
        jax.ShapeDtypeStruct((_B, _D), jnp.float32),
        jax.ShapeDtypeStruct((_B, _D), jnp.float32),
    ),
    mesh=plsc.VectorSubcoreMesh(core_axis_name="c", subcore_axis_name="s"),
    scratch_types=[
        pltpu.VMEM((2 * _BPW,), jnp.int32),
        pltpu.VMEM((_BPW,), jnp.int32),
        pltpu.VMEM((_BPW,), jnp.int32),
        pltpu.VMEM((_BPW, _D), jnp.float32),
        pltpu.VMEM((_BPW, _D), jnp.float32),
        pltpu.SemaphoreType.DMA,
    ],
)


def kernel(entity, symbol_emb):
    idx = entity.astype(jnp.int32).reshape(_NW, 2 * _BPW)
    return _gather(idx, symbol_emb)


# speedup vs baseline: 1.0192x; 1.0192x over previous
import jax
import jax.numpy as jnp
from jax import lax
from jax.experimental import pallas as pl
from jax.experimental.pallas import tpu as pltpu
from jax.experimental.pallas import tpu_sc as plsc

_B = 4096
_D = 128
_NC = 2
_NS = 16
_NW = _NC * _NS
_ROWS = 2 * _B
_RPW = _ROWS // _NW
_CHUNK = 128
_NCHUNK = _RPW // _CHUNK


def _body(idx_hbm, table_hbm, left_hbm, right_hbm, idx_v, rows_v, gsem, wsem):
    wid = lax.axis_index("s") * _NC + lax.axis_index("c")
    pltpu.sync_copy(idx_hbm.at[wid], idx_v)
    gathers = [
        pltpu.async_copy(
            table_hbm.at[idx_v.at[c]],
            rows_v.at[pl.ds(c * _CHUNK, _CHUNK)],
            gsem,
        )
        for c in range(_NCHUNK)
    ]

    half = _NW // 2

    def drain(out_hbm, base):
        writes = []
        for c in range(_NCHUNK):
            gathers[c].wait()
            writes.append(pltpu.async_copy(
                rows_v.at[pl.ds(c * _CHUNK, _CHUNK)],
                out_hbm.at[pl.ds(base + c * _CHUNK, _CHUNK)],
                wsem,
            ))
        for w in writes:
            w.wait()

    @pl.when(wid < half)
    def _():
        drain(left_hbm, wid * _RPW)

    @pl.when(wid >= half)
    def _():
        drain(right_hbm, (wid - half) * _RPW)


_gather = pl.kernel(
    _body,
    out_type=(
        jax.ShapeDtypeStruct((_B, _D), jnp.float32),
        jax.ShapeDtypeStruct((_B, _D), jnp.float32),
    ),
    mesh=plsc.VectorSubcoreMesh(core_axis_name="c", subcore_axis_name="s"),
    scratch_types=[
        pltpu.VMEM((_NCHUNK, _CHUNK), jnp.int32),
        pltpu.VMEM((_RPW, _D), jnp.float32),
        pltpu.SemaphoreType.DMA,
        pltpu.SemaphoreType.DMA,
    ],
)


def kernel(entity, symbol_emb):
    idx = entity.astype(jnp.int32).T.reshape(_NW, _NCHUNK, _CHUNK)
    return _gather(idx, symbol_emb)
